# baseline (speedup 1.0000x reference)
import jax
import jax.numpy as jnp
from jax import lax
from jax.experimental import pallas as pl
from jax.experimental.pallas import tpu as pltpu
from jax.experimental.pallas import tpu_sc as plsc

N = 10000
E = 160000
D_IN = 256
D_HID = 256
D_OUT = 64

NC = 2
NS = 16
DC = D_IN // NC

N_PAD = N + NS
BA = 128

SG0 = 2
ITS0 = 40

SG1 = 2
ITS1 = 20

ROWS_PT = 624
ROWS_TAIL = N - NS * ROWS_PT

_mesh = plsc.VectorSubcoreMesh(core_axis_name="c", subcore_axis_name="s")


def _rows_copy(src_ref, dst_ref, s):
    pltpu.sync_copy(src_ref.at[pl.ds(s * ROWS_PT, ROWS_PT)],
                    dst_ref.at[pl.ds(s * ROWS_PT, ROWS_PT)])

    @pl.when(s == 0)
    def _():
        pltpu.sync_copy(src_ref.at[pl.ds(NS * ROWS_PT, ROWS_TAIL)],
                        dst_ref.at[pl.ds(NS * ROWS_PT, ROWS_TAIL)])


def _edge_pipeline(x_ref, src_st, dst_st, src_sp, dst_sp, rows_v,
                   sem, acc_sh, stages, its, deg=None):
    ones_v, deg_sh = deg if deg is not None else (None, None)

    for st in range(stages):
        pltpu.sync_copy(src_st.at[st], src_sp)
        pltpu.sync_copy(dst_st.at[st], dst_sp)

        def body(i, carry):
            pltpu.async_copy(x_ref.at[src_sp.at[i]], rows_v, sem).wait()
            pltpu.sync_copy(rows_v, acc_sh.at[dst_sp.at[i]], add=True)
            if deg is not None:
                pltpu.sync_copy(ones_v, deg_sh.at[dst_sp.at[i]], add=True)
            return carry
        lax.fori_loop(0, its, body, 0)


def _spmm0_body(x_lo, x_hi, src_a, dst_a, zcol, zdeg,
                acc_lo_out, acc_hi_out, deg_out,
                acc_sh, deg_sh, src_sp, dst_sp, rows_v, ones_v, sem):
    c = lax.axis_index("c")
    s = lax.axis_index("s")
    _rows_copy(zcol, acc_sh, s)

    @pl.when(jnp.logical_and(c == 0, s == 0))
    def _():
        pltpu.sync_copy(zcol.at[pl.ds(N, N_PAD - N)],
                        acc_sh.at[pl.ds(N, N_PAD - N)])
        pltpu.sync_copy(zdeg, deg_sh)

    @pl.when(jnp.logical_and(c == 1, s == 0))
    def _():
        pltpu.sync_copy(zcol.at[pl.ds(N, N_PAD - N)],
                        acc_sh.at[pl.ds(N, N_PAD - N)])

    for j in range(BA // 16):
        ones_v[pl.ds(j * 16, 16)] = jnp.ones((16,), jnp.float32)
    plsc.subcore_barrier()

    @pl.when(c == 0)
    def _():
        _edge_pipeline(x_lo, src_a.at[s], dst_a.at[s], src_sp, dst_sp,
                       rows_v, sem, acc_sh, SG0, ITS0,
                       deg=(ones_v, deg_sh))

    @pl.when(c == 1)
    def _():
        _edge_pipeline(x_hi, src_a.at[s], dst_a.at[s], src_sp, dst_sp,
                       rows_v, sem, acc_sh, SG0, ITS0)

    plsc.subcore_barrier()

    @pl.when(c == 0)
    def _():
        _rows_copy(acc_sh, acc_lo_out, s)

        @pl.when(s == 0)
        def _():
            pltpu.sync_copy(deg_sh, deg_out)

    @pl.when(c == 1)
    def _():
        _rows_copy(acc_sh, acc_hi_out, s)


_spmm0 = pl.kernel(
    _spmm0_body,
    out_type=[
        jax.ShapeDtypeStruct((N, DC), jnp.float32),
        jax.ShapeDtypeStruct((N, DC), jnp.float32),
        jax.ShapeDtypeStruct((N_PAD,), jnp.float32),
    ],
    mesh=_mesh,
    scratch_types=[
        pltpu.VMEM_SHARED((N_PAD, DC), jnp.float32),
        pltpu.VMEM_SHARED((N_PAD,), jnp.float32),
        pltpu.VMEM((ITS0, BA), jnp.int32),
        pltpu.VMEM((ITS0, BA), jnp.int32),
        pltpu.VMEM((BA, DC), jnp.float32),
        pltpu.VMEM((BA,), jnp.float32),
        pltpu.SemaphoreType.DMA,
    ],
)


def _spmm1_body(t_h, src_b, dst_b, zcol, p0_out, p1_out,
                acc_sh, src_sp, dst_sp, rows_v, sem):
    c = lax.axis_index("c")
    s = lax.axis_index("s")
    w = c * NS + s
    _rows_copy(zcol, acc_sh, s)

    @pl.when(s == 0)
    def _():
        pltpu.sync_copy(zcol.at[pl.ds(N, N_PAD - N)],
                        acc_sh.at[pl.ds(N, N_PAD - N)])

    plsc.subcore_barrier()

    _edge_pipeline(t_h, src_b.at[w], dst_b.at[w], src_sp, dst_sp,
                   rows_v, sem, acc_sh, SG1, ITS1)

    plsc.subcore_barrier()

    @pl.when(c == 0)
    def _():
        _rows_copy(acc_sh, p0_out, s)

    @pl.when(c == 1)
    def _():
        _rows_copy(acc_sh, p1_out, s)


_spmm1 = pl.kernel(
    _spmm1_body,
    out_type=[
        jax.ShapeDtypeStruct((N, DC), jnp.float32),
        jax.ShapeDtypeStruct((N, DC), jnp.float32),
    ],
    mesh=_mesh,
    scratch_types=[
        pltpu.VMEM_SHARED((N_PAD, DC), jnp.float32),
        pltpu.VMEM((ITS1, BA), jnp.int32),
        pltpu.VMEM((ITS1, BA), jnp.int32),
        pltpu.VMEM((BA, DC), jnp.float32),
        pltpu.SemaphoreType.DMA,
    ],
)

_R = 1000


def _dense0_body(alo, ahi, deg, x, wl0, bl0, wr0, wl1, act1_o, t_o):
    d = jnp.maximum(deg[...], 1.0)
    w0 = wl0[...]
    z = (jnp.dot(alo[...] / d, w0[:DC, :], preferred_element_type=jnp.float32)
         + jnp.dot(ahi[...] / d, w0[DC:, :], preferred_element_type=jnp.float32)
         + jnp.dot(x[...], wr0[...], preferred_element_type=jnp.float32)
         + bl0[...])
    a = jnp.maximum(z, 0.0)
    act1_o[...] = a
    t = jnp.dot(a, wl1[...], preferred_element_type=jnp.float32)
    t_o[...] = jnp.concatenate(
        [t, jnp.zeros((t.shape[0], DC - D_OUT), jnp.float32)], axis=1)


def _dense0(acc_lo, acc_hi, deg2, x, wl0, bl0, wr0, wl1):
    grid = (N // _R,)
    return pl.pallas_call(
        _dense0_body,
        grid=grid,
        in_specs=[
            pl.BlockSpec((_R, DC), lambda i: (i, 0)),
            pl.BlockSpec((_R, DC), lambda i: (i, 0)),
            pl.BlockSpec((_R, 1), lambda i: (i, 0)),
            pl.BlockSpec((_R, D_IN), lambda i: (i, 0)),
            pl.BlockSpec((D_IN, D_HID), lambda i: (0, 0)),
            pl.BlockSpec((1, D_HID), lambda i: (0, 0)),
            pl.BlockSpec((D_IN, D_HID), lambda i: (0, 0)),
            pl.BlockSpec((D_HID, D_OUT), lambda i: (0, 0)),
        ],
        out_specs=[
            pl.BlockSpec((_R, D_HID), lambda i: (i, 0)),
            pl.BlockSpec((_R, DC), lambda i: (i, 0)),
        ],
        out_shape=[
            jax.ShapeDtypeStruct((N, D_HID), jnp.float32),
            jax.ShapeDtypeStruct((N, DC), jnp.float32),
        ],
    )(acc_lo, acc_hi, deg2, x, wl0, bl0, wr0, wl1)


def _dense1_body(p0, p1, deg, act1, wr1, bl1, out_o):
    d = jnp.maximum(deg[...], 1.0)
    z = ((p0[...][:, :D_OUT] + p1[...][:, :D_OUT]) / d
         + jnp.dot(act1[...], wr1[...], preferred_element_type=jnp.float32)
         + bl1[...])
    m = jnp.max(z, axis=1, keepdims=True)
    ez = jnp.exp(z - m)
    lse = jnp.log(jnp.sum(ez, axis=1, keepdims=True)) + m
    out_o[...] = z - lse


def _dense1(p0, p1, deg2, act1, wr1, bl1):
    grid = (N // _R,)
    return pl.pallas_call(
        _dense1_body,
        grid=grid,
        in_specs=[
            pl.BlockSpec((_R, DC), lambda i: (i, 0)),
            pl.BlockSpec((_R, DC), lambda i: (i, 0)),
            pl.BlockSpec((_R, 1), lambda i: (i, 0)),
            pl.BlockSpec((_R, D_HID), lambda i: (i, 0)),
            pl.BlockSpec((D_HID, D_OUT), lambda i: (0, 0)),
            pl.BlockSpec((1, D_OUT), lambda i: (0, 0)),
        ],
        out_specs=pl.BlockSpec((_R, D_OUT), lambda i: (i, 0)),
        out_shape=jax.ShapeDtypeStruct((N, D_OUT), jnp.float32),
    )(p0, p1, deg2, act1, wr1, bl1)


def _pad_edges(a, n_chunks, stages, its, dummy_row):
    per = E // n_chunks
    per_pad = stages * its * BA
    a2 = a.reshape(n_chunks, per)
    if dummy_row:
        dummy = N + (jnp.arange(n_chunks, dtype=jnp.int32) % NS)[:, None]
        pad = jnp.broadcast_to(dummy, (n_chunks, per_pad - per))
    else:
        pad = jnp.zeros((n_chunks, per_pad - per), jnp.int32)
    return jnp.concatenate([a2, pad], axis=1).reshape(
        n_chunks, stages, its, BA)


def kernel(x, adj, default_chunk_size, chunk_sizes_diff,
           W_l0, b_l0, W_r0, W_l1, b_l1, W_r1):
    dst = adj[0].astype(jnp.int32)
    src = adj[1].astype(jnp.int32)
    src_a = _pad_edges(src, NS, SG0, ITS0, False)
    dst_a = _pad_edges(dst, NS, SG0, ITS0, True)
    src_b = _pad_edges(src, NC * NS, SG1, ITS1, False)
    dst_b = _pad_edges(dst, NC * NS, SG1, ITS1, True)
    x_lo = x[:, :DC]
    x_hi = x[:, DC:]
    zcol = jnp.zeros((N_PAD, DC), jnp.float32)
    zdeg = jnp.zeros((N_PAD,), jnp.float32)

    acc_lo, acc_hi, deg = _spmm0(x_lo, x_hi, src_a, dst_a, zcol, zdeg)
    deg2 = deg[:N].reshape(N, 1)
    act1, t = _dense0(acc_lo, acc_hi, deg2, x, W_l0,
                      b_l0.reshape(1, -1), W_r0, W_l1)
    p0, p1 = _spmm1(t, src_b, dst_b, zcol)
    return _dense1(p0, p1, deg2, act1, W_r1, b_l1.reshape(1, -1))


# speedup vs baseline: 1.4374x; 1.4374x over previous
import jax
import jax.numpy as jnp
from jax import lax
from jax.experimental import pallas as pl
from jax.experimental.pallas import tpu as pltpu
from jax.experimental.pallas import tpu_sc as plsc

N = 10000
E = 160000
D_IN = 256
D_HID = 256
D_OUT = 64

NC = 2
NS = 16
DC = D_IN // NC

EA_PT = E // NS
BA = 80
ITA = EA_PT // BA

EB_PT = E // (NC * NS)
BB = 40
ITB = EB_PT // BB

ROWS_PT = 624
ROWS_TAIL = N - NS * ROWS_PT

_mesh = plsc.VectorSubcoreMesh(core_axis_name="c", subcore_axis_name="s")


def _rows_copy(src_ref, dst_ref, s):
    pltpu.sync_copy(src_ref.at[pl.ds(s * ROWS_PT, ROWS_PT)],
                    dst_ref.at[pl.ds(s * ROWS_PT, ROWS_PT)])

    @pl.when(s == 0)
    def _():
        pltpu.sync_copy(src_ref.at[pl.ds(NS * ROWS_PT, ROWS_TAIL)],
                        dst_ref.at[pl.ds(NS * ROWS_PT, ROWS_TAIL)])


def _spmm0_body(x_lo, x_hi, src_a, dst_a, zcol, zdeg,
                acc_lo_out, acc_hi_out, deg_out,
                acc_sh, deg_sh, src_sp, dst_sp, rows_v, ones_v, sem):
    c = lax.axis_index("c")
    s = lax.axis_index("s")
    _rows_copy(zcol, acc_sh, s)

    @pl.when(jnp.logical_and(c == 0, s == 0))
    def _():
        pltpu.sync_copy(zdeg, deg_sh)

    pltpu.sync_copy(src_a.at[s], src_sp)
    pltpu.sync_copy(dst_a.at[s], dst_sp)
    for j in range(BA // 16):
        ones_v[pl.ds(j * 16, 16)] = jnp.ones((16,), jnp.float32)
    plsc.subcore_barrier()

    def edge_loop(x_ref, count_deg):
        def body(i, carry):
            pltpu.async_copy(x_ref.at[src_sp.at[i]], rows_v, sem).wait()
            pltpu.sync_copy(rows_v, acc_sh.at[dst_sp.at[i]], add=True)
            if count_deg:
                pltpu.sync_copy(ones_v, deg_sh.at[dst_sp.at[i]], add=True)
            return carry
        lax.fori_loop(0, ITA, body, 0)

    @pl.when(c == 0)
    def _():
        edge_loop(x_lo, True)

    @pl.when(c == 1)
    def _():
        edge_loop(x_hi, False)

    plsc.subcore_barrier()

    @pl.when(c == 0)
    def _():
        _rows_copy(acc_sh, acc_lo_out, s)

        @pl.when(s == 0)
        def _():
            pltpu.sync_copy(deg_sh, deg_out)

    @pl.when(c == 1)
    def _():
        _rows_copy(acc_sh, acc_hi_out, s)


_spmm0 = pl.kernel(
    _spmm0_body,
    out_type=[
        jax.ShapeDtypeStruct((N, DC), jnp.float32),
        jax.ShapeDtypeStruct((N, DC), jnp.float32),
        jax.ShapeDtypeStruct((N,), jnp.float32),
    ],
    mesh=_mesh,
    scratch_types=[
        pltpu.VMEM_SHARED((N, DC), jnp.float32),
        pltpu.VMEM_SHARED((N,), jnp.float32),
        pltpu.VMEM((ITA, BA), jnp.int32),
        pltpu.VMEM((ITA, BA), jnp.int32),
        pltpu.VMEM((BA, DC), jnp.float32),
        pltpu.VMEM((BA,), jnp.float32),
        pltpu.SemaphoreType.DMA,
    ],
)


def _spmm1_body(t_h, src_b, dst_b, zcol2, p0_out, p1_out,
                acc_sh, src_sp, dst_sp, rows_v, sem):
    c = lax.axis_index("c")
    s = lax.axis_index("s")
    w = c * NS + s
    _rows_copy(zcol2, acc_sh, s)
    pltpu.sync_copy(src_b.at[w], src_sp)
    pltpu.sync_copy(dst_b.at[w], dst_sp)
    plsc.subcore_barrier()

    def body(i, carry):
        pltpu.async_copy(t_h.at[src_sp.at[i]], rows_v, sem).wait()
        pltpu.sync_copy(rows_v, acc_sh.at[dst_sp.at[i]], add=True)
        return carry
    lax.fori_loop(0, ITB, body, 0)

    plsc.subcore_barrier()

    @pl.when(c == 0)
    def _():
        _rows_copy(acc_sh, p0_out, s)

    @pl.when(c == 1)
    def _():
        _rows_copy(acc_sh, p1_out, s)


_spmm1 = pl.kernel(
    _spmm1_body,
    out_type=[
        jax.ShapeDtypeStruct((N, DC), jnp.float32),
        jax.ShapeDtypeStruct((N, DC), jnp.float32),
    ],
    mesh=_mesh,
    scratch_types=[
        pltpu.VMEM_SHARED((N, DC), jnp.float32),
        pltpu.VMEM((ITB, BB), jnp.int32),
        pltpu.VMEM((ITB, BB), jnp.int32),
        pltpu.VMEM((BB, DC), jnp.float32),
        pltpu.SemaphoreType.DMA,
    ],
)

_R = 1000


def _dense0_body(alo, ahi, deg, x, wl0, bl0, wr0, wl1, act1_o, t_o):
    d = jnp.maximum(deg[...], 1.0)
    w0 = wl0[...]
    z = (jnp.dot(alo[...] / d, w0[:DC, :], preferred_element_type=jnp.float32)
         + jnp.dot(ahi[...] / d, w0[DC:, :], preferred_element_type=jnp.float32)
         + jnp.dot(x[...], wr0[...], preferred_element_type=jnp.float32)
         + bl0[...])
    a = jnp.maximum(z, 0.0)
    act1_o[...] = a
    t = jnp.dot(a, wl1[...], preferred_element_type=jnp.float32)
    t_o[...] = jnp.concatenate(
        [t, jnp.zeros((t.shape[0], DC - D_OUT), jnp.float32)], axis=1)


def _dense0(acc_lo, acc_hi, deg2, x, wl0, bl0, wr0, wl1):
    grid = (N // _R,)
    return pl.pallas_call(
        _dense0_body,
        grid=grid,
        in_specs=[
            pl.BlockSpec((_R, DC), lambda i: (i, 0)),
            pl.BlockSpec((_R, DC), lambda i: (i, 0)),
            pl.BlockSpec((_R, 1), lambda i: (i, 0)),
            pl.BlockSpec((_R, D_IN), lambda i: (i, 0)),
            pl.BlockSpec((D_IN, D_HID), lambda i: (0, 0)),
            pl.BlockSpec((1, D_HID), lambda i: (0, 0)),
            pl.BlockSpec((D_IN, D_HID), lambda i: (0, 0)),
            pl.BlockSpec((D_HID, D_OUT), lambda i: (0, 0)),
        ],
        out_specs=[
            pl.BlockSpec((_R, D_HID), lambda i: (i, 0)),
            pl.BlockSpec((_R, DC), lambda i: (i, 0)),
        ],
        out_shape=[
            jax.ShapeDtypeStruct((N, D_HID), jnp.float32),
            jax.ShapeDtypeStruct((N, DC), jnp.float32),
        ],
    )(acc_lo, acc_hi, deg2, x, wl0, bl0, wr0, wl1)


def _dense1_body(p0, p1, deg, act1, wr1, bl1, out_o):
    d = jnp.maximum(deg[...], 1.0)
    z = ((p0[...][:, :D_OUT] + p1[...][:, :D_OUT]) / d
         + jnp.dot(act1[...], wr1[...], preferred_element_type=jnp.float32)
         + bl1[...])
    m = jnp.max(z, axis=1, keepdims=True)
    ez = jnp.exp(z - m)
    lse = jnp.log(jnp.sum(ez, axis=1, keepdims=True)) + m
    out_o[...] = z - lse


def _dense1(p0, p1, deg2, act1, wr1, bl1):
    grid = (N // _R,)
    return pl.pallas_call(
        _dense1_body,
        grid=grid,
        in_specs=[
            pl.BlockSpec((_R, DC), lambda i: (i, 0)),
            pl.BlockSpec((_R, DC), lambda i: (i, 0)),
            pl.BlockSpec((_R, 1), lambda i: (i, 0)),
            pl.BlockSpec((_R, D_HID), lambda i: (i, 0)),
            pl.BlockSpec((D_HID, D_OUT), lambda i: (0, 0)),
            pl.BlockSpec((1, D_OUT), lambda i: (0, 0)),
        ],
        out_specs=pl.BlockSpec((_R, D_OUT), lambda i: (i, 0)),
        out_shape=jax.ShapeDtypeStruct((N, D_OUT), jnp.float32),
    )(p0, p1, deg2, act1, wr1, bl1)


def kernel(x, adj, default_chunk_size, chunk_sizes_diff,
           W_l0, b_l0, W_r0, W_l1, b_l1, W_r1):
    dst = adj[0].astype(jnp.int32)
    src = adj[1].astype(jnp.int32)
    src_a = src.reshape(NS, ITA, BA)
    dst_a = dst.reshape(NS, ITA, BA)
    src_b = src.reshape(NC * NS, ITB, BB)
    dst_b = dst.reshape(NC * NS, ITB, BB)
    x_lo = x[:, :DC]
    x_hi = x[:, DC:]
    zcol = jnp.zeros((N, DC), jnp.float32)
    zdeg = jnp.zeros((N,), jnp.float32)

    acc_lo, acc_hi, deg = _spmm0(x_lo, x_hi, src_a, dst_a, zcol, zdeg)
    deg2 = deg.reshape(N, 1)
    act1, t = _dense0(acc_lo, acc_hi, deg2, x, W_l0,
                      b_l0.reshape(1, -1), W_r0, W_l1)
    p0, p1 = _spmm1(t, src_b, dst_b, zcol)
    return _dense1(p0, p1, deg2, act1, W_r1, b_l1.reshape(1, -1))


# speedup vs baseline: 1.6194x; 1.1266x over previous
import jax
import jax.numpy as jnp
from jax import lax
from jax.experimental import pallas as pl
from jax.experimental.pallas import tpu as pltpu
from jax.experimental.pallas import tpu_sc as plsc

N = 10000
E = 160000
D_IN = 256
D_HID = 256
D_OUT = 64

NC = 2
NS = 16
DC = D_IN // NC

EA_PT = E // NS
BA = 80
ITA = EA_PT // BA

EB_PT = E // (NC * NS)
BB = 100
ITB = EB_PT // BB

ROWS_PT = 624
ROWS_TAIL = N - NS * ROWS_PT

_mesh = plsc.VectorSubcoreMesh(core_axis_name="c", subcore_axis_name="s")


def _rows_copy(src_ref, dst_ref, s):
    pltpu.sync_copy(src_ref.at[pl.ds(s * ROWS_PT, ROWS_PT)],
                    dst_ref.at[pl.ds(s * ROWS_PT, ROWS_PT)])

    @pl.when(s == 0)
    def _():
        pltpu.sync_copy(src_ref.at[pl.ds(NS * ROWS_PT, ROWS_TAIL)],
                        dst_ref.at[pl.ds(NS * ROWS_PT, ROWS_TAIL)])


def _spmm0_body(x_lo, x_hi, src_a, dst_a, zcol, zdeg,
                acc_lo_out, acc_hi_out, deg_out,
                acc_sh, deg_sh, src_sp, dst_sp, rows_v, ones_v, sem):
    c = lax.axis_index("c")
    s = lax.axis_index("s")
    _rows_copy(zcol, acc_sh, s)

    @pl.when(jnp.logical_and(c == 0, s == 0))
    def _():
        pltpu.sync_copy(zdeg, deg_sh)

    pltpu.sync_copy(src_a.at[s], src_sp)
    pltpu.sync_copy(dst_a.at[s], dst_sp)
    for j in range(BA // 16):
        ones_v[pl.ds(j * 16, 16)] = jnp.ones((16,), jnp.float32)
    plsc.subcore_barrier()

    def edge_loop(x_ref, count_deg):
        def body(i, carry):
            pltpu.async_copy(x_ref.at[src_sp.at[i]], rows_v, sem).wait()
            pltpu.sync_copy(rows_v, acc_sh.at[dst_sp.at[i]], add=True)
            if count_deg:
                pltpu.sync_copy(ones_v, deg_sh.at[dst_sp.at[i]], add=True)
            return carry
        lax.fori_loop(0, ITA, body, 0)

    @pl.when(c == 0)
    def _():
        edge_loop(x_lo, True)

    @pl.when(c == 1)
    def _():
        edge_loop(x_hi, False)

    plsc.subcore_barrier()

    @pl.when(c == 0)
    def _():
        _rows_copy(acc_sh, acc_lo_out, s)

        @pl.when(s == 0)
        def _():
            pltpu.sync_copy(deg_sh, deg_out)

    @pl.when(c == 1)
    def _():
        _rows_copy(acc_sh, acc_hi_out, s)


_spmm0 = pl.kernel(
    _spmm0_body,
    out_type=[
        jax.ShapeDtypeStruct((N, DC), jnp.float32),
        jax.ShapeDtypeStruct((N, DC), jnp.float32),
        jax.ShapeDtypeStruct((N,), jnp.float32),
    ],
    mesh=_mesh,
    scratch_types=[
        pltpu.VMEM_SHARED((N, DC), jnp.float32),
        pltpu.VMEM_SHARED((N,), jnp.float32),
        pltpu.VMEM((ITA, BA), jnp.int32),
        pltpu.VMEM((ITA, BA), jnp.int32),
        pltpu.VMEM((BA, DC), jnp.float32),
        pltpu.VMEM((BA,), jnp.float32),
        pltpu.SemaphoreType.DMA,
    ],
)


def _spmm1_body(t_h, src_b, dst_b, zcol2, p0_out, p1_out,
                acc_sh, src_sp, dst_sp, rows_v, sem):
    c = lax.axis_index("c")
    s = lax.axis_index("s")
    w = c * NS + s
    _rows_copy(zcol2, acc_sh, s)
    pltpu.sync_copy(src_b.at[w], src_sp)
    pltpu.sync_copy(dst_b.at[w], dst_sp)
    plsc.subcore_barrier()

    def body(i, carry):
        pltpu.async_copy(t_h.at[src_sp.at[i]], rows_v, sem).wait()
        pltpu.sync_copy(rows_v, acc_sh.at[dst_sp.at[i]], add=True)
        return carry
    lax.fori_loop(0, ITB, body, 0)

    plsc.subcore_barrier()

    @pl.when(c == 0)
    def _():
        _rows_copy(acc_sh, p0_out, s)

    @pl.when(c == 1)
    def _():
        _rows_copy(acc_sh, p1_out, s)


_spmm1 = pl.kernel(
    _spmm1_body,
    out_type=[
        jax.ShapeDtypeStruct((N, DC), jnp.float32),
        jax.ShapeDtypeStruct((N, DC), jnp.float32),
    ],
    mesh=_mesh,
    scratch_types=[
        pltpu.VMEM_SHARED((N, DC), jnp.float32),
        pltpu.VMEM((ITB, BB), jnp.int32),
        pltpu.VMEM((ITB, BB), jnp.int32),
        pltpu.VMEM((BB, DC), jnp.float32),
        pltpu.SemaphoreType.DMA,
    ],
)

_R = 1000


def _dense0_body(alo, ahi, deg, x, wl0, bl0, wr0, wl1, act1_o, t_o):
    d = jnp.maximum(deg[...], 1.0)
    w0 = wl0[...]
    z = (jnp.dot(alo[...] / d, w0[:DC, :], preferred_element_type=jnp.float32)
         + jnp.dot(ahi[...] / d, w0[DC:, :], preferred_element_type=jnp.float32)
         + jnp.dot(x[...], wr0[...], preferred_element_type=jnp.float32)
         + bl0[...])
    a = jnp.maximum(z, 0.0)
    act1_o[...] = a
    t = jnp.dot(a, wl1[...], preferred_element_type=jnp.float32)
    t_o[...] = jnp.concatenate(
        [t, jnp.zeros((t.shape[0], DC - D_OUT), jnp.float32)], axis=1)


def _dense0(acc_lo, acc_hi, deg2, x, wl0, bl0, wr0, wl1):
    grid = (N // _R,)
    return pl.pallas_call(
        _dense0_body,
        grid=grid,
        in_specs=[
            pl.BlockSpec((_R, DC), lambda i: (i, 0)),
            pl.BlockSpec((_R, DC), lambda i: (i, 0)),
            pl.BlockSpec((_R, 1), lambda i: (i, 0)),
            pl.BlockSpec((_R, D_IN), lambda i: (i, 0)),
            pl.BlockSpec((D_IN, D_HID), lambda i: (0, 0)),
            pl.BlockSpec((1, D_HID), lambda i: (0, 0)),
            pl.BlockSpec((D_IN, D_HID), lambda i: (0, 0)),
            pl.BlockSpec((D_HID, D_OUT), lambda i: (0, 0)),
        ],
        out_specs=[
            pl.BlockSpec((_R, D_HID), lambda i: (i, 0)),
            pl.BlockSpec((_R, DC), lambda i: (i, 0)),
        ],
        out_shape=[
            jax.ShapeDtypeStruct((N, D_HID), jnp.float32),
            jax.ShapeDtypeStruct((N, DC), jnp.float32),
        ],
    )(acc_lo, acc_hi, deg2, x, wl0, bl0, wr0, wl1)


def _dense1_body(p0, p1, deg, act1, wr1, bl1, out_o):
    d = jnp.maximum(deg[...], 1.0)
    z = ((p0[...][:, :D_OUT] + p1[...][:, :D_OUT]) / d
         + jnp.dot(act1[...], wr1[...], preferred_element_type=jnp.float32)
         + bl1[...])
    m = jnp.max(z, axis=1, keepdims=True)
    ez = jnp.exp(z - m)
    lse = jnp.log(jnp.sum(ez, axis=1, keepdims=True)) + m
    out_o[...] = z - lse


def _dense1(p0, p1, deg2, act1, wr1, bl1):
    grid = (N // _R,)
    return pl.pallas_call(
        _dense1_body,
        grid=grid,
        in_specs=[
            pl.BlockSpec((_R, DC), lambda i: (i, 0)),
            pl.BlockSpec((_R, DC), lambda i: (i, 0)),
            pl.BlockSpec((_R, 1), lambda i: (i, 0)),
            pl.BlockSpec((_R, D_HID), lambda i: (i, 0)),
            pl.BlockSpec((D_HID, D_OUT), lambda i: (0, 0)),
            pl.BlockSpec((1, D_OUT), lambda i: (0, 0)),
        ],
        out_specs=pl.BlockSpec((_R, D_OUT), lambda i: (i, 0)),
        out_shape=jax.ShapeDtypeStruct((N, D_OUT), jnp.float32),
    )(p0, p1, deg2, act1, wr1, bl1)


def kernel(x, adj, default_chunk_size, chunk_sizes_diff,
           W_l0, b_l0, W_r0, W_l1, b_l1, W_r1):
    dst = adj[0].astype(jnp.int32)
    src = adj[1].astype(jnp.int32)
    src_a = src.reshape(NS, ITA, BA)
    dst_a = dst.reshape(NS, ITA, BA)
    src_b = src.reshape(NC * NS, ITB, BB)
    dst_b = dst.reshape(NC * NS, ITB, BB)
    x_lo = x[:, :DC]
    x_hi = x[:, DC:]
    zcol = jnp.zeros((N, DC), jnp.float32)
    zdeg = jnp.zeros((N,), jnp.float32)

    acc_lo, acc_hi, deg = _spmm0(x_lo, x_hi, src_a, dst_a, zcol, zdeg)
    deg2 = deg.reshape(N, 1)
    act1, t = _dense0(acc_lo, acc_hi, deg2, x, W_l0,
                      b_l0.reshape(1, -1), W_r0, W_l1)
    p0, p1 = _spmm1(t, src_b, dst_b, zcol)
    return _dense1(p0, p1, deg2, act1, W_r1, b_l1.reshape(1, -1))


# speedup vs baseline: 1.7025x; 1.0514x over previous
import jax
import jax.numpy as jnp
from jax import lax
from jax.experimental import pallas as pl
from jax.experimental.pallas import tpu as pltpu
from jax.experimental.pallas import tpu_sc as plsc

N = 10000
E = 160000
D_IN = 256
D_HID = 256
D_OUT = 64

NC = 2
NS = 16
DC = D_IN // NC

EA_PT = E // NS
BA = 100
ITA = EA_PT // BA

EB_PT = E // (NC * NS)
BB = 100
ITB = EB_PT // BB

ROWS_PT = 624
ROWS_TAIL = N - NS * ROWS_PT
_ONES_PAD = ((BA + 15) // 16) * 16

_mesh = plsc.VectorSubcoreMesh(core_axis_name="c", subcore_axis_name="s")


def _rows_copy(src_ref, dst_ref, s):
    pltpu.sync_copy(src_ref.at[pl.ds(s * ROWS_PT, ROWS_PT)],
                    dst_ref.at[pl.ds(s * ROWS_PT, ROWS_PT)])

    @pl.when(s == 0)
    def _():
        pltpu.sync_copy(src_ref.at[pl.ds(NS * ROWS_PT, ROWS_TAIL)],
                        dst_ref.at[pl.ds(NS * ROWS_PT, ROWS_TAIL)])


def _spmm0_body(x_lo, x_hi, src_a, dst_a, zcol, zdeg,
                acc_lo_out, acc_hi_out, deg_out,
                acc_sh, deg_sh, src_sp, dst_sp, rows_v, ones_v, sem):
    c = lax.axis_index("c")
    s = lax.axis_index("s")
    _rows_copy(zcol, acc_sh, s)

    @pl.when(jnp.logical_and(c == 0, s == 0))
    def _():
        pltpu.sync_copy(zdeg, deg_sh)

    pltpu.sync_copy(src_a.at[s], src_sp)
    pltpu.sync_copy(dst_a.at[s], dst_sp)
    for j in range(_ONES_PAD // 16):
        ones_v[pl.ds(j * 16, 16)] = jnp.ones((16,), jnp.float32)
    plsc.subcore_barrier()

    def edge_loop(x_ref, count_deg):
        def body(i, carry):
            pltpu.async_copy(x_ref.at[src_sp.at[i]], rows_v, sem).wait()
            pltpu.sync_copy(rows_v, acc_sh.at[dst_sp.at[i]], add=True)
            if count_deg:
                pltpu.sync_copy(ones_v.at[pl.ds(0, BA)],
                                deg_sh.at[dst_sp.at[i]], add=True)
            return carry
        lax.fori_loop(0, ITA, body, 0)

    @pl.when(c == 0)
    def _():
        edge_loop(x_lo, True)

    @pl.when(c == 1)
    def _():
        edge_loop(x_hi, False)

    plsc.subcore_barrier()

    @pl.when(c == 0)
    def _():
        _rows_copy(acc_sh, acc_lo_out, s)

        @pl.when(s == 0)
        def _():
            pltpu.sync_copy(deg_sh, deg_out)

    @pl.when(c == 1)
    def _():
        _rows_copy(acc_sh, acc_hi_out, s)


_spmm0 = pl.kernel(
    _spmm0_body,
    out_type=[
        jax.ShapeDtypeStruct((N, DC), jnp.float32),
        jax.ShapeDtypeStruct((N, DC), jnp.float32),
        jax.ShapeDtypeStruct((N,), jnp.float32),
    ],
    mesh=_mesh,
    scratch_types=[
        pltpu.VMEM_SHARED((N, DC), jnp.float32),
        pltpu.VMEM_SHARED((N,), jnp.float32),
        pltpu.VMEM((ITA, BA), jnp.int32),
        pltpu.VMEM((ITA, BA), jnp.int32),
        pltpu.VMEM((BA, DC), jnp.float32),
        pltpu.VMEM((_ONES_PAD,), jnp.float32),
        pltpu.SemaphoreType.DMA,
    ],
)


def _spmm1_body(t_h, src_b, dst_b, zcol2, p0_out, p1_out,
                acc_sh, src_sp, dst_sp, rows_v, sem):
    c = lax.axis_index("c")
    s = lax.axis_index("s")
    w = c * NS + s
    _rows_copy(zcol2, acc_sh, s)
    pltpu.sync_copy(src_b.at[w], src_sp)
    pltpu.sync_copy(dst_b.at[w], dst_sp)
    plsc.subcore_barrier()

    def body(i, carry):
        pltpu.async_copy(t_h.at[src_sp.at[i]], rows_v, sem).wait()
        pltpu.sync_copy(rows_v, acc_sh.at[dst_sp.at[i]], add=True)
        return carry
    lax.fori_loop(0, ITB, body, 0)

    plsc.subcore_barrier()

    @pl.when(c == 0)
    def _():
        _rows_copy(acc_sh, p0_out, s)

    @pl.when(c == 1)
    def _():
        _rows_copy(acc_sh, p1_out, s)


_spmm1 = pl.kernel(
    _spmm1_body,
    out_type=[
        jax.ShapeDtypeStruct((N, DC), jnp.float32),
        jax.ShapeDtypeStruct((N, DC), jnp.float32),
    ],
    mesh=_mesh,
    scratch_types=[
        pltpu.VMEM_SHARED((N, DC), jnp.float32),
        pltpu.VMEM((ITB, BB), jnp.int32),
        pltpu.VMEM((ITB, BB), jnp.int32),
        pltpu.VMEM((BB, DC), jnp.float32),
        pltpu.SemaphoreType.DMA,
    ],
)

_R = 1000


def _dense0_body(alo, ahi, deg, x, wl0, bl0, wr0, wl1, act1_o, t_o):
    d = jnp.maximum(deg[...], 1.0)
    w0 = wl0[...]
    z = (jnp.dot(alo[...] / d, w0[:DC, :], preferred_element_type=jnp.float32)
         + jnp.dot(ahi[...] / d, w0[DC:, :], preferred_element_type=jnp.float32)
         + jnp.dot(x[...], wr0[...], preferred_element_type=jnp.float32)
         + bl0[...])
    a = jnp.maximum(z, 0.0)
    act1_o[...] = a
    t = jnp.dot(a, wl1[...], preferred_element_type=jnp.float32)
    t_o[...] = jnp.concatenate(
        [t, jnp.zeros((t.shape[0], DC - D_OUT), jnp.float32)], axis=1)


def _dense0(acc_lo, acc_hi, deg2, x, wl0, bl0, wr0, wl1):
    grid = (N // _R,)
    return pl.pallas_call(
        _dense0_body,
        grid=grid,
        in_specs=[
            pl.BlockSpec((_R, DC), lambda i: (i, 0)),
            pl.BlockSpec((_R, DC), lambda i: (i, 0)),
            pl.BlockSpec((_R, 1), lambda i: (i, 0)),
            pl.BlockSpec((_R, D_IN), lambda i: (i, 0)),
            pl.BlockSpec((D_IN, D_HID), lambda i: (0, 0)),
            pl.BlockSpec((1, D_HID), lambda i: (0, 0)),
            pl.BlockSpec((D_IN, D_HID), lambda i: (0, 0)),
            pl.BlockSpec((D_HID, D_OUT), lambda i: (0, 0)),
        ],
        out_specs=[
            pl.BlockSpec((_R, D_HID), lambda i: (i, 0)),
            pl.BlockSpec((_R, DC), lambda i: (i, 0)),
        ],
        out_shape=[
            jax.ShapeDtypeStruct((N, D_HID), jnp.float32),
            jax.ShapeDtypeStruct((N, DC), jnp.float32),
        ],
    )(acc_lo, acc_hi, deg2, x, wl0, bl0, wr0, wl1)


def _dense1_body(p0, p1, deg, act1, wr1, bl1, out_o):
    d = jnp.maximum(deg[...], 1.0)
    z = ((p0[...][:, :D_OUT] + p1[...][:, :D_OUT]) / d
         + jnp.dot(act1[...], wr1[...], preferred_element_type=jnp.float32)
         + bl1[...])
    m = jnp.max(z, axis=1, keepdims=True)
    ez = jnp.exp(z - m)
    lse = jnp.log(jnp.sum(ez, axis=1, keepdims=True)) + m
    out_o[...] = z - lse


def _dense1(p0, p1, deg2, act1, wr1, bl1):
    grid = (N // _R,)
    return pl.pallas_call(
        _dense1_body,
        grid=grid,
        in_specs=[
            pl.BlockSpec((_R, DC), lambda i: (i, 0)),
            pl.BlockSpec((_R, DC), lambda i: (i, 0)),
            pl.BlockSpec((_R, 1), lambda i: (i, 0)),
            pl.BlockSpec((_R, D_HID), lambda i: (i, 0)),
            pl.BlockSpec((D_HID, D_OUT), lambda i: (0, 0)),
            pl.BlockSpec((1, D_OUT), lambda i: (0, 0)),
        ],
        out_specs=pl.BlockSpec((_R, D_OUT), lambda i: (i, 0)),
        out_shape=jax.ShapeDtypeStruct((N, D_OUT), jnp.float32),
    )(p0, p1, deg2, act1, wr1, bl1)


def kernel(x, adj, default_chunk_size, chunk_sizes_diff,
           W_l0, b_l0, W_r0, W_l1, b_l1, W_r1):
    dst = adj[0].astype(jnp.int32)
    src = adj[1].astype(jnp.int32)
    src_a = src.reshape(NS, ITA, BA)
    dst_a = dst.reshape(NS, ITA, BA)
    src_b = src.reshape(NC * NS, ITB, BB)
    dst_b = dst.reshape(NC * NS, ITB, BB)
    x_lo = x[:, :DC]
    x_hi = x[:, DC:]
    zcol = jnp.zeros((N, DC), jnp.float32)
    zdeg = jnp.zeros((N,), jnp.float32)

    acc_lo, acc_hi, deg = _spmm0(x_lo, x_hi, src_a, dst_a, zcol, zdeg)
    deg2 = deg.reshape(N, 1)
    act1, t = _dense0(acc_lo, acc_hi, deg2, x, W_l0,
                      b_l0.reshape(1, -1), W_r0, W_l1)
    p0, p1 = _spmm1(t, src_b, dst_b, zcol)
    return _dense1(p0, p1, deg2, act1, W_r1, b_l1.reshape(1, -1))


# speedup vs baseline: 1.7984x; 1.0563x over previous
import jax
import jax.numpy as jnp
from jax import lax
from jax.experimental import pallas as pl
from jax.experimental.pallas import tpu as pltpu
from jax.experimental.pallas import tpu_sc as plsc

N = 10000
E = 160000
D_IN = 256
D_HID = 256
D_OUT = 64

NC = 2
NS = 16
DC = D_IN // NC

EA_PT = E // NS
BA = 125
ITA = EA_PT // BA

EB_PT = E // (NC * NS)
BB = 125
ITB = EB_PT // BB

ROWS_PT = 624
ROWS_TAIL = N - NS * ROWS_PT
_ONES_PAD = ((BA + 15) // 16) * 16

_mesh = plsc.VectorSubcoreMesh(core_axis_name="c", subcore_axis_name="s")


def _rows_copy(src_ref, dst_ref, s):
    pltpu.sync_copy(src_ref.at[pl.ds(s * ROWS_PT, ROWS_PT)],
                    dst_ref.at[pl.ds(s * ROWS_PT, ROWS_PT)])

    @pl.when(s == 0)
    def _():
        pltpu.sync_copy(src_ref.at[pl.ds(NS * ROWS_PT, ROWS_TAIL)],
                        dst_ref.at[pl.ds(NS * ROWS_PT, ROWS_TAIL)])


def _spmm0_body(x_lo, x_hi, src_a, dst_a, zcol, zdeg,
                acc_lo_out, acc_hi_out, deg_out,
                acc_sh, deg_sh, src_sp, dst_sp, rows_v, ones_v, sem):
    c = lax.axis_index("c")
    s = lax.axis_index("s")
    _rows_copy(zcol, acc_sh, s)

    @pl.when(jnp.logical_and(c == 0, s == 0))
    def _():
        pltpu.sync_copy(zdeg, deg_sh)

    pltpu.sync_copy(src_a.at[s], src_sp)
    pltpu.sync_copy(dst_a.at[s], dst_sp)
    for j in range(_ONES_PAD // 16):
        ones_v[pl.ds(j * 16, 16)] = jnp.ones((16,), jnp.float32)
    plsc.subcore_barrier()

    def edge_loop(x_ref, count_deg):
        def body(i, carry):
            pltpu.async_copy(x_ref.at[src_sp.at[i]], rows_v, sem).wait()
            pltpu.sync_copy(rows_v, acc_sh.at[dst_sp.at[i]], add=True)
            if count_deg:
                pltpu.sync_copy(ones_v.at[pl.ds(0, BA)],
                                deg_sh.at[dst_sp.at[i]], add=True)
            return carry
        lax.fori_loop(0, ITA, body, 0)

    @pl.when(c == 0)
    def _():
        edge_loop(x_lo, True)

    @pl.when(c == 1)
    def _():
        edge_loop(x_hi, False)

    plsc.subcore_barrier()

    @pl.when(c == 0)
    def _():
        _rows_copy(acc_sh, acc_lo_out, s)

        @pl.when(s == 0)
        def _():
            pltpu.sync_copy(deg_sh, deg_out)

    @pl.when(c == 1)
    def _():
        _rows_copy(acc_sh, acc_hi_out, s)


_spmm0 = pl.kernel(
    _spmm0_body,
    out_type=[
        jax.ShapeDtypeStruct((N, DC), jnp.float32),
        jax.ShapeDtypeStruct((N, DC), jnp.float32),
        jax.ShapeDtypeStruct((N,), jnp.float32),
    ],
    mesh=_mesh,
    scratch_types=[
        pltpu.VMEM_SHARED((N, DC), jnp.float32),
        pltpu.VMEM_SHARED((N,), jnp.float32),
        pltpu.VMEM((ITA, BA), jnp.int32),
        pltpu.VMEM((ITA, BA), jnp.int32),
        pltpu.VMEM((BA, DC), jnp.float32),
        pltpu.VMEM((_ONES_PAD,), jnp.float32),
        pltpu.SemaphoreType.DMA,
    ],
)


def _spmm1_body(t_h, src_b, dst_b, zcol2, p0_out, p1_out,
                acc_sh, src_sp, dst_sp, rows_v, sem):
    c = lax.axis_index("c")
    s = lax.axis_index("s")
    w = c * NS + s
    _rows_copy(zcol2, acc_sh, s)
    pltpu.sync_copy(src_b.at[w], src_sp)
    pltpu.sync_copy(dst_b.at[w], dst_sp)
    plsc.subcore_barrier()

    def body(i, carry):
        pltpu.async_copy(t_h.at[src_sp.at[i]], rows_v, sem).wait()
        pltpu.sync_copy(rows_v, acc_sh.at[dst_sp.at[i]], add=True)
        return carry
    lax.fori_loop(0, ITB, body, 0)

    plsc.subcore_barrier()

    @pl.when(c == 0)
    def _():
        _rows_copy(acc_sh, p0_out, s)

    @pl.when(c == 1)
    def _():
        _rows_copy(acc_sh, p1_out, s)


_spmm1 = pl.kernel(
    _spmm1_body,
    out_type=[
        jax.ShapeDtypeStruct((N, DC), jnp.float32),
        jax.ShapeDtypeStruct((N, DC), jnp.float32),
    ],
    mesh=_mesh,
    scratch_types=[
        pltpu.VMEM_SHARED((N, DC), jnp.float32),
        pltpu.VMEM((ITB, BB), jnp.int32),
        pltpu.VMEM((ITB, BB), jnp.int32),
        pltpu.VMEM((BB, DC), jnp.float32),
        pltpu.SemaphoreType.DMA,
    ],
)

_R = 1000


def _dense0_body(alo, ahi, deg, x, wl0, bl0, wr0, wl1, act1_o, t_o):
    d = jnp.maximum(deg[...], 1.0)
    w0 = wl0[...]
    z = (jnp.dot(alo[...] / d, w0[:DC, :], preferred_element_type=jnp.float32)
         + jnp.dot(ahi[...] / d, w0[DC:, :], preferred_element_type=jnp.float32)
         + jnp.dot(x[...], wr0[...], preferred_element_type=jnp.float32)
         + bl0[...])
    a = jnp.maximum(z, 0.0)
    act1_o[...] = a
    t = jnp.dot(a, wl1[...], preferred_element_type=jnp.float32)
    t_o[...] = jnp.concatenate(
        [t, jnp.zeros((t.shape[0], DC - D_OUT), jnp.float32)], axis=1)


def _dense0(acc_lo, acc_hi, deg2, x, wl0, bl0, wr0, wl1):
    grid = (N // _R,)
    return pl.pallas_call(
        _dense0_body,
        grid=grid,
        in_specs=[
            pl.BlockSpec((_R, DC), lambda i: (i, 0)),
            pl.BlockSpec((_R, DC), lambda i: (i, 0)),
            pl.BlockSpec((_R, 1), lambda i: (i, 0)),
            pl.BlockSpec((_R, D_IN), lambda i: (i, 0)),
            pl.BlockSpec((D_IN, D_HID), lambda i: (0, 0)),
            pl.BlockSpec((1, D_HID), lambda i: (0, 0)),
            pl.BlockSpec((D_IN, D_HID), lambda i: (0, 0)),
            pl.BlockSpec((D_HID, D_OUT), lambda i: (0, 0)),
        ],
        out_specs=[
            pl.BlockSpec((_R, D_HID), lambda i: (i, 0)),
            pl.BlockSpec((_R, DC), lambda i: (i, 0)),
        ],
        out_shape=[
            jax.ShapeDtypeStruct((N, D_HID), jnp.float32),
            jax.ShapeDtypeStruct((N, DC), jnp.float32),
        ],
    )(acc_lo, acc_hi, deg2, x, wl0, bl0, wr0, wl1)


def _dense1_body(p0, p1, deg, act1, wr1, bl1, out_o):
    d = jnp.maximum(deg[...], 1.0)
    z = ((p0[...][:, :D_OUT] + p1[...][:, :D_OUT]) / d
         + jnp.dot(act1[...], wr1[...], preferred_element_type=jnp.float32)
         + bl1[...])
    m = jnp.max(z, axis=1, keepdims=True)
    ez = jnp.exp(z - m)
    lse = jnp.log(jnp.sum(ez, axis=1, keepdims=True)) + m
    out_o[...] = z - lse


def _dense1(p0, p1, deg2, act1, wr1, bl1):
    grid = (N // _R,)
    return pl.pallas_call(
        _dense1_body,
        grid=grid,
        in_specs=[
            pl.BlockSpec((_R, DC), lambda i: (i, 0)),
            pl.BlockSpec((_R, DC), lambda i: (i, 0)),
            pl.BlockSpec((_R, 1), lambda i: (i, 0)),
            pl.BlockSpec((_R, D_HID), lambda i: (i, 0)),
            pl.BlockSpec((D_HID, D_OUT), lambda i: (0, 0)),
            pl.BlockSpec((1, D_OUT), lambda i: (0, 0)),
        ],
        out_specs=pl.BlockSpec((_R, D_OUT), lambda i: (i, 0)),
        out_shape=jax.ShapeDtypeStruct((N, D_OUT), jnp.float32),
    )(p0, p1, deg2, act1, wr1, bl1)


def kernel(x, adj, default_chunk_size, chunk_sizes_diff,
           W_l0, b_l0, W_r0, W_l1, b_l1, W_r1):
    dst = adj[0].astype(jnp.int32)
    src = adj[1].astype(jnp.int32)
    src_a = src.reshape(NS, ITA, BA)
    dst_a = dst.reshape(NS, ITA, BA)
    src_b = src.reshape(NC * NS, ITB, BB)
    dst_b = dst.reshape(NC * NS, ITB, BB)
    x_lo = x[:, :DC]
    x_hi = x[:, DC:]
    zcol = jnp.zeros((N, DC), jnp.float32)
    zdeg = jnp.zeros((N,), jnp.float32)

    acc_lo, acc_hi, deg = _spmm0(x_lo, x_hi, src_a, dst_a, zcol, zdeg)
    deg2 = deg.reshape(N, 1)
    act1, t = _dense0(acc_lo, acc_hi, deg2, x, W_l0,
                      b_l0.reshape(1, -1), W_r0, W_l1)
    p0, p1 = _spmm1(t, src_b, dst_b, zcol)
    return _dense1(p0, p1, deg2, act1, W_r1, b_l1.reshape(1, -1))
